# baseline (speedup 1.0000x reference)
import functools

import jax
import jax.numpy as jnp
from jax import lax
from jax.experimental import pallas as pl
from jax.experimental.pallas import tpu as pltpu
from jax.experimental.pallas import tpu_sc as plsc

DIM = 32
LANES = 16
CHUNK = 128


def _make_scorer(batch, dim):
    info = plsc.get_sparse_core_info()
    nc, ns = info.num_cores, info.num_subcores
    nw = nc * ns
    b_per_w = batch // nw
    n_chunks = b_per_w // CHUNK
    n_groups = b_per_w // LANES

    mesh = plsc.VectorSubcoreMesh(core_axis_name="c", subcore_axis_name="s")

    @functools.partial(
        pl.kernel,
        mesh=mesh,
        out_type=jax.ShapeDtypeStruct((batch,), jnp.float32),
        compiler_params=pltpu.CompilerParams(needs_layout_passes=False,
                                             use_tc_tiling_on_sc=False),
        scratch_types=[
            pltpu.VMEM((n_chunks, CHUNK), jnp.int32),
            pltpu.VMEM((n_chunks, CHUNK), jnp.int32),
            pltpu.VMEM((b_per_w, dim), jnp.float32),
            pltpu.VMEM((b_per_w, dim), jnp.float32),
            pltpu.VMEM((b_per_w,), jnp.float32),
            pltpu.VMEM((LANES * LANES,), jnp.float32),
            pltpu.SemaphoreType.DMA,
        ],
    )
    def scorer(u_idx_hbm, i_idx_hbm, user_hbm, item_hbm, out_hbm,
               u_idx_v, i_idx_v, u_rows, i_rows, out_v, tr_v, sem):
        wid = lax.axis_index("s") * nc + lax.axis_index("c")
        base = wid * b_per_w

        for c in range(n_chunks):
            pltpu.sync_copy(u_idx_hbm.at[pl.ds(base + c * CHUNK, CHUNK)],
                            u_idx_v.at[c])
            pltpu.sync_copy(i_idx_hbm.at[pl.ds(base + c * CHUNK, CHUNK)],
                            i_idx_v.at[c])

        copies = []
        for c in range(n_chunks):
            copies.append(pltpu.async_copy(
                user_hbm.at[u_idx_v.at[c]],
                u_rows.at[pl.ds(c * CHUNK, CHUNK)], sem))
            copies.append(pltpu.async_copy(
                item_hbm.at[i_idx_v.at[c]],
                i_rows.at[pl.ds(c * CHUNK, CHUNK)], sem))
        for cp in copies:
            cp.wait()

        lanes = lax.iota(jnp.int32, LANES)
        tr_base = lanes * LANES

        def group_body(g, _):
            base_r = g * LANES
            for l in range(LANES):
                r = base_r + l
                part = (u_rows[r, pl.ds(0, LANES)] * i_rows[r, pl.ds(0, LANES)]
                        + u_rows[r, pl.ds(LANES, LANES)]
                        * i_rows[r, pl.ds(LANES, LANES)])
                plsc.store_scatter(tr_v, [tr_base + l], part)
            acc = tr_v[pl.ds(0, LANES)]
            for k in range(1, LANES):
                acc = acc + tr_v[pl.ds(k * LANES, LANES)]
            out_v[pl.ds(base_r, LANES)] = acc
            return _

        lax.fori_loop(0, n_groups, group_body, None)

        pltpu.sync_copy(out_v, out_hbm.at[pl.ds(base, b_per_w)])

    return scorer


def kernel(u_idx, i_idx, user_emb, item_emb):
    batch = u_idx.shape[0]
    scorer = _make_scorer(batch, user_emb.shape[1])
    return scorer(u_idx.astype(jnp.int32), i_idx.astype(jnp.int32),
                  user_emb, item_emb)


# speedup vs baseline: 3.9423x; 3.9423x over previous
import functools

import jax
import jax.numpy as jnp
from jax import lax
from jax.experimental import pallas as pl
from jax.experimental.pallas import tpu as pltpu
from jax.experimental.pallas import tpu_sc as plsc

LANES = 16
RING = 8


def _make_scorer(batch, dim):
    info = plsc.get_sparse_core_info()
    nc, ns = info.num_cores, info.num_subcores
    nw = nc * ns
    b_per_w = batch // nw
    n_groups = b_per_w // LANES

    mesh = plsc.VectorSubcoreMesh(core_axis_name="c", subcore_axis_name="s")

    @functools.partial(
        pl.kernel,
        mesh=mesh,
        out_type=jax.ShapeDtypeStruct((batch,), jnp.float32),
        compiler_params=pltpu.CompilerParams(needs_layout_passes=False),
        scratch_types=(
            [
                pltpu.VMEM((b_per_w,), jnp.int32),
                pltpu.VMEM((b_per_w,), jnp.int32),
                pltpu.VMEM((RING, dim, 128), jnp.float32),
                pltpu.VMEM((RING, dim, 128), jnp.float32),
                pltpu.VMEM((b_per_w,), jnp.float32),
                pltpu.VMEM((LANES * LANES,), jnp.float32),
            ]
            + [pltpu.SemaphoreType.DMA] * (2 * RING)
        ),
    )
    def scorer(u_idx_hbm, i_idx_hbm, ut_hbm, it_hbm, out_hbm,
               u_idx_v, i_idx_v, u_ring, i_ring, out_v, tr_v, *sems):
        u_sems, i_sems = sems[:RING], sems[RING:]
        wid = lax.axis_index("s") * nc + lax.axis_index("c")
        base = wid * b_per_w

        pltpu.sync_copy(u_idx_hbm.at[pl.ds(base, b_per_w)], u_idx_v)
        pltpu.sync_copy(i_idx_hbm.at[pl.ds(base, b_per_w)], i_idx_v)

        lanes = lax.iota(jnp.int32, LANES)
        d_lo = lanes
        d_hi = lanes + LANES
        tr_base = lanes * LANES
        zeros = jnp.zeros((LANES,), jnp.int32)

        def extract(vec, lane):
            return jnp.sum(jnp.where(lanes == lane, vec, zeros))

        def fire(xu, xi, b):
            qu = pl.multiple_of((xu >> 7) * 128, 128)
            pltpu.async_copy(ut_hbm.at[:, pl.ds(qu, 128)], u_ring.at[b],
                             u_sems[b])
            qi = pl.multiple_of((xi >> 7) * 128, 128)
            pltpu.async_copy(it_hbm.at[:, pl.ds(qi, 128)], i_ring.at[b],
                             i_sems[b])

        uvec0 = u_idx_v[pl.ds(0, LANES)]
        ivec0 = i_idx_v[pl.ds(0, LANES)]
        for b in range(RING):
            fire(extract(uvec0, b), extract(ivec0, b), b)

        def outer(g, _):
            uvec = u_idx_v[pl.ds(g * LANES, LANES)]
            ivec = i_idx_v[pl.ds(g * LANES, LANES)]
            for t in range(LANES):
                b = t % RING
                pltpu.make_async_copy(
                    ut_hbm.at[:, pl.ds(0, 128)], u_ring.at[b],
                    u_sems[b]).wait()
                pltpu.make_async_copy(
                    it_hbm.at[:, pl.ds(0, 128)], i_ring.at[b],
                    i_sems[b]).wait()
                xu = extract(uvec, t)
                xi = extract(ivec, t)
                b_vec = jnp.full((LANES,), b, jnp.int32)
                xu_vec = jnp.full((LANES,), xu & 127, jnp.int32)
                xi_vec = jnp.full((LANES,), xi & 127, jnp.int32)
                u0 = plsc.load_gather(u_ring, [b_vec, d_lo, xu_vec])
                u1 = plsc.load_gather(u_ring, [b_vec, d_hi, xu_vec])
                i0 = plsc.load_gather(i_ring, [b_vec, d_lo, xi_vec])
                i1 = plsc.load_gather(i_ring, [b_vec, d_hi, xi_vec])
                plsc.store_scatter(tr_v, [tr_base + t], u0 * i0 + u1 * i1)

                if t < LANES - RING:
                    fire(extract(uvec, t + RING), extract(ivec, t + RING), b)
                else:
                    @pl.when(g < n_groups - 1)
                    def _refire():
                        uvec2 = u_idx_v[pl.ds((g + 1) * LANES, LANES)]
                        ivec2 = i_idx_v[pl.ds((g + 1) * LANES, LANES)]
                        fire(extract(uvec2, t + RING - LANES),
                             extract(ivec2, t + RING - LANES), b)
            acc = tr_v[pl.ds(0, LANES)]
            for k in range(1, LANES):
                acc = acc + tr_v[pl.ds(k * LANES, LANES)]
            out_v[pl.ds(g * LANES, LANES)] = acc
            return _

        lax.fori_loop(0, n_groups, outer, None)

        pltpu.sync_copy(out_v, out_hbm.at[pl.ds(base, b_per_w)])

    return scorer


def kernel(u_idx, i_idx, user_emb, item_emb):
    batch = u_idx.shape[0]
    scorer = _make_scorer(batch, user_emb.shape[1])
    return scorer(u_idx.astype(jnp.int32), i_idx.astype(jnp.int32),
                  user_emb.T, item_emb.T)
